# Initial kernel scaffold; baseline (speedup 1.0000x reference)
import jax
import jax.numpy as jnp
from jax.experimental import pallas as pl


def kernel(x, embed):
    raise NotImplementedError("write your pallas kernel here")


# speedup vs baseline: 1.3092x; 1.3092x over previous
import functools

import jax
import jax.numpy as jnp
from jax import lax
from jax.experimental import pallas as pl
from jax.experimental.pallas import tpu as pltpu
from jax.experimental.pallas import tpu_sc as plsc

_K = 8192
_D = 256
_BN = 512
_KT = 2736


def _argmin_body(xn_ref, en_ref, x_ref, e_ref, ind_ref):
    x = x_ref[...]
    xn = xn_ref[...]
    acc_v = None
    acc_i = None
    for lo in range(0, _K, _KT):
        w = min(_KT, _K - lo)
        e_t = e_ref[pl.ds(lo, w), :]
        scores = lax.dot_general(
            x, e_t, (((1,), (1,)), ((), ())),
            preferred_element_type=jnp.float32)
        t2 = (xn - 2.0 * scores) + en_ref[:, pl.ds(lo, w)]
        tmin = jnp.min(t2, axis=1, keepdims=True)
        gidx = lax.broadcasted_iota(jnp.int32, t2.shape, 1) + lo
        cand = jnp.min(jnp.where(t2 == tmin, gidx, jnp.int32(2**30)),
                       axis=1, keepdims=True)
        if acc_v is None:
            acc_v, acc_i = tmin, cand
        else:
            take = tmin < acc_v
            acc_v = jnp.where(take, tmin, acc_v)
            acc_i = jnp.where(take, cand, acc_i)
        acc_v = acc_v.astype(jnp.bfloat16).astype(jnp.float32)
    ind_ref[...] = acc_i


def _nearest_code(x_flat, xn, en, embed):
    n = x_flat.shape[0]
    return pl.pallas_call(
        _argmin_body,
        grid=(n // _BN,),
        in_specs=[
            pl.BlockSpec((_BN, 1), lambda i: (i, 0)),
            pl.BlockSpec((1, _K), lambda i: (0, 0)),
            pl.BlockSpec((_BN, _D), lambda i: (i, 0)),
            pl.BlockSpec((_K, _D), lambda i: (0, 0)),
        ],
        out_specs=pl.BlockSpec((_BN, 1), lambda i: (i, 0)),
        out_shape=jax.ShapeDtypeStruct((n, 1), jnp.int32),
        compiler_params=pltpu.CompilerParams(
            dimension_semantics=("arbitrary",)),
    )(xn, en, x_flat, embed)


_GCH = 128


def _sc_gather(table, idx):
    n = idx.shape[0]
    info = plsc.get_sparse_core_info()
    nw = info.num_cores * info.num_subcores
    per_w = n // nw
    mesh = plsc.VectorSubcoreMesh(core_axis_name="c", subcore_axis_name="s")

    @functools.partial(
        pl.kernel, mesh=mesh,
        out_type=jax.ShapeDtypeStruct((n, _D), table.dtype),
        scratch_types=[
            pltpu.VMEM((_GCH,), jnp.int32),
            pltpu.VMEM((_GCH, _D), jnp.float32),
            pltpu.SemaphoreType.DMA,
        ],
    )
    def gk(table_hbm, idx_hbm, out_hbm, idx_v, rows_v, sem):
        wid = lax.axis_index("s") * info.num_cores + lax.axis_index("c")
        base = wid * per_w

        @pl.loop(0, per_w // _GCH)
        def _(c):
            off = base + c * _GCH
            pltpu.sync_copy(idx_hbm.at[pl.ds(off, _GCH)], idx_v)
            pltpu.async_copy(table_hbm.at[idx_v], rows_v, sem).wait()
            pltpu.sync_copy(rows_v, out_hbm.at[pl.ds(off, _GCH)])

    return gk(table, idx)


def kernel(x, embed):
    shape = x.shape
    x_flat = x.reshape(-1, shape[-1])
    xn = jnp.sum(x_flat ** 2, axis=1, keepdims=True)
    embed_t = embed.T
    en = jnp.sum(embed_t ** 2, axis=0, keepdims=True)
    ind_flat = _nearest_code(x_flat, xn, en, embed).reshape(-1)
    quantized = _sc_gather(embed, ind_flat).reshape(shape)
    embed_ind = ind_flat.reshape(shape[:-1])
    return quantized, embed_ind


# speedup vs baseline: 1.4729x; 1.1251x over previous
import functools

import jax
import jax.numpy as jnp
from jax import lax
from jax.experimental import pallas as pl
from jax.experimental.pallas import tpu as pltpu
from jax.experimental.pallas import tpu_sc as plsc

_K = 8192
_D = 256
_BN = 512
_KT = 2736


def _argmin_body(xn_ref, en_ref, x_ref, e_ref, ind_ref):
    x = x_ref[...]
    xn = xn_ref[...]
    acc_v = None
    acc_i = None
    for lo in range(0, _K, _KT):
        w = min(_KT, _K - lo)
        e_t = e_ref[pl.ds(lo, w), :]
        scores = lax.dot_general(
            x, e_t, (((1,), (1,)), ((), ())),
            preferred_element_type=jnp.float32)
        t2 = (xn - 2.0 * scores) + en_ref[:, pl.ds(lo, w)]
        tmin = jnp.min(t2, axis=1, keepdims=True)
        fidx = lax.broadcasted_iota(jnp.int32, t2.shape, 1).astype(jnp.float32)
        cand = jnp.min(jnp.where(t2 == tmin, fidx, jnp.float32(3e7)),
                       axis=1, keepdims=True) + jnp.float32(lo)
        if acc_v is None:
            acc_v, acc_i = tmin, cand
        else:
            take = tmin < acc_v
            acc_v = jnp.where(take, tmin, acc_v)
            acc_i = jnp.where(take, cand, acc_i)
        acc_v = acc_v.astype(jnp.bfloat16).astype(jnp.float32)
    ind_ref[...] = acc_i.astype(jnp.int32)


def _nearest_code(x_flat, xn, en, embed):
    n = x_flat.shape[0]
    return pl.pallas_call(
        _argmin_body,
        grid=(n // _BN,),
        in_specs=[
            pl.BlockSpec((_BN, 1), lambda i: (i, 0)),
            pl.BlockSpec((1, _K), lambda i: (0, 0)),
            pl.BlockSpec((_BN, _D), lambda i: (i, 0)),
            pl.BlockSpec((_K, _D), lambda i: (0, 0)),
        ],
        out_specs=pl.BlockSpec((_BN, 1), lambda i: (i, 0)),
        out_shape=jax.ShapeDtypeStruct((n, 1), jnp.int32),
        compiler_params=pltpu.CompilerParams(
            dimension_semantics=("arbitrary",)),
    )(xn, en, x_flat, embed)


_GCH = 128


def _sc_gather(table, idx):
    n = idx.shape[0]
    info = plsc.get_sparse_core_info()
    nw = info.num_cores * info.num_subcores
    per_w = n // nw
    mesh = plsc.VectorSubcoreMesh(core_axis_name="c", subcore_axis_name="s")

    @functools.partial(
        pl.kernel, mesh=mesh,
        out_type=jax.ShapeDtypeStruct((n, _D), table.dtype),
        scratch_types=[
            pltpu.VMEM((_GCH,), jnp.int32),
            pltpu.VMEM((_GCH, _D), jnp.float32),
            pltpu.SemaphoreType.DMA,
        ],
    )
    def gk(table_hbm, idx_hbm, out_hbm, idx_v, rows_v, sem):
        wid = lax.axis_index("s") * info.num_cores + lax.axis_index("c")
        base = wid * per_w

        @pl.loop(0, per_w // _GCH)
        def _(c):
            off = base + c * _GCH
            pltpu.sync_copy(idx_hbm.at[pl.ds(off, _GCH)], idx_v)
            pltpu.async_copy(table_hbm.at[idx_v], rows_v, sem).wait()
            pltpu.sync_copy(rows_v, out_hbm.at[pl.ds(off, _GCH)])

    return gk(table, idx)


def kernel(x, embed):
    shape = x.shape
    x_flat = x.reshape(-1, shape[-1])
    xn = jnp.sum(x_flat ** 2, axis=1, keepdims=True)
    embed_t = embed.T
    en = jnp.sum(embed_t ** 2, axis=0, keepdims=True)
    ind_flat = _nearest_code(x_flat, xn, en, embed).reshape(-1)
    quantized = _sc_gather(embed, ind_flat).reshape(shape)
    embed_ind = ind_flat.reshape(shape[:-1])
    return quantized, embed_ind


# speedup vs baseline: 1.5789x; 1.0719x over previous
import functools

import jax
import jax.numpy as jnp
from jax import lax
from jax.experimental import pallas as pl
from jax.experimental.pallas import tpu as pltpu
from jax.experimental.pallas import tpu_sc as plsc

_K = 8192
_D = 256
_BN = 1024
_KT = 2736


def _argmin_body(xn_ref, en_ref, x_ref, e_ref, ind_ref):
    x = x_ref[...]
    xn = xn_ref[...]
    acc_v = None
    acc_i = None
    for lo in range(0, _K, _KT):
        w = min(_KT, _K - lo)
        e_t = e_ref[pl.ds(lo, w), :]
        scores = lax.dot_general(
            x, e_t, (((1,), (1,)), ((), ())),
            preferred_element_type=jnp.float32)
        t2 = (xn - 2.0 * scores) + en_ref[:, pl.ds(lo, w)]
        tmin = jnp.min(t2, axis=1, keepdims=True)
        fidx = lax.broadcasted_iota(jnp.int32, t2.shape, 1).astype(jnp.float32)
        cand = jnp.min(jnp.where(t2 == tmin, fidx, jnp.float32(3e7)),
                       axis=1, keepdims=True) + jnp.float32(lo)
        if acc_v is None:
            acc_v, acc_i = tmin, cand
        else:
            take = tmin < acc_v
            acc_v = jnp.where(take, tmin, acc_v)
            acc_i = jnp.where(take, cand, acc_i)
        acc_v = acc_v.astype(jnp.bfloat16).astype(jnp.float32)
    ind_ref[...] = acc_i.astype(jnp.int32)


def _nearest_code(x_flat, xn, en, embed):
    n = x_flat.shape[0]
    return pl.pallas_call(
        _argmin_body,
        grid=(n // _BN,),
        in_specs=[
            pl.BlockSpec((_BN, 1), lambda i: (i, 0)),
            pl.BlockSpec((1, _K), lambda i: (0, 0)),
            pl.BlockSpec((_BN, _D), lambda i: (i, 0)),
            pl.BlockSpec((_K, _D), lambda i: (0, 0)),
        ],
        out_specs=pl.BlockSpec((_BN, 1), lambda i: (i, 0)),
        out_shape=jax.ShapeDtypeStruct((n, 1), jnp.int32),
        compiler_params=pltpu.CompilerParams(
            dimension_semantics=("arbitrary",)),
    )(xn, en, x_flat, embed)


_GCH = 128


def _sc_gather(table, idx):
    n = idx.shape[0]
    info = plsc.get_sparse_core_info()
    nw = info.num_cores * info.num_subcores
    per_w = n // nw
    mesh = plsc.VectorSubcoreMesh(core_axis_name="c", subcore_axis_name="s")

    @functools.partial(
        pl.kernel, mesh=mesh,
        out_type=jax.ShapeDtypeStruct((n, _D), table.dtype),
        scratch_types=[
            pltpu.VMEM((_GCH,), jnp.int32),
            pltpu.VMEM((_GCH, _D), jnp.float32),
            pltpu.SemaphoreType.DMA,
        ],
    )
    def gk(table_hbm, idx_hbm, out_hbm, idx_v, rows_v, sem):
        wid = lax.axis_index("s") * info.num_cores + lax.axis_index("c")
        base = wid * per_w

        @pl.loop(0, per_w // _GCH)
        def _(c):
            off = base + c * _GCH
            pltpu.sync_copy(idx_hbm.at[pl.ds(off, _GCH)], idx_v)
            pltpu.async_copy(table_hbm.at[idx_v], rows_v, sem).wait()
            pltpu.sync_copy(rows_v, out_hbm.at[pl.ds(off, _GCH)])

    return gk(table, idx)


def kernel(x, embed):
    shape = x.shape
    x_flat = x.reshape(-1, shape[-1])
    xn = jnp.sum(x_flat ** 2, axis=1, keepdims=True)
    embed_t = embed.T
    en = jnp.sum(embed_t ** 2, axis=0, keepdims=True)
    ind_flat = _nearest_code(x_flat, xn, en, embed).reshape(-1)
    quantized = _sc_gather(embed, ind_flat).reshape(shape)
    embed_ind = ind_flat.reshape(shape[:-1])
    return quantized, embed_ind


# speedup vs baseline: 1.6084x; 1.0187x over previous
import functools

import jax
import jax.numpy as jnp
from jax import lax
from jax.experimental import pallas as pl
from jax.experimental.pallas import tpu as pltpu
from jax.experimental.pallas import tpu_sc as plsc

_K = 8192
_D = 256
_BN = 2048
_KT = 2736


def _argmin_body(xn_ref, en_ref, x_ref, e_ref, ind_ref):
    x = x_ref[...]
    xn = xn_ref[...]
    acc_v = None
    acc_i = None
    for lo in range(0, _K, _KT):
        w = min(_KT, _K - lo)
        e_t = e_ref[pl.ds(lo, w), :]
        scores = lax.dot_general(
            x, e_t, (((1,), (1,)), ((), ())),
            preferred_element_type=jnp.float32)
        t2 = (xn - 2.0 * scores) + en_ref[:, pl.ds(lo, w)]
        tmin = jnp.min(t2, axis=1, keepdims=True)
        fidx = lax.broadcasted_iota(jnp.int32, t2.shape, 1).astype(jnp.float32)
        cand = jnp.min(jnp.where(t2 == tmin, fidx, jnp.float32(3e7)),
                       axis=1, keepdims=True) + jnp.float32(lo)
        if acc_v is None:
            acc_v, acc_i = tmin, cand
        else:
            take = tmin < acc_v
            acc_v = jnp.where(take, tmin, acc_v)
            acc_i = jnp.where(take, cand, acc_i)
        acc_v = acc_v.astype(jnp.bfloat16).astype(jnp.float32)
    ind_ref[...] = acc_i.astype(jnp.int32)


def _nearest_code(x_flat, xn, en, embed):
    n = x_flat.shape[0]
    return pl.pallas_call(
        _argmin_body,
        grid=(n // _BN,),
        in_specs=[
            pl.BlockSpec((_BN, 1), lambda i: (i, 0)),
            pl.BlockSpec((1, _K), lambda i: (0, 0)),
            pl.BlockSpec((_BN, _D), lambda i: (i, 0)),
            pl.BlockSpec((_K, _D), lambda i: (0, 0)),
        ],
        out_specs=pl.BlockSpec((_BN, 1), lambda i: (i, 0)),
        out_shape=jax.ShapeDtypeStruct((n, 1), jnp.int32),
        compiler_params=pltpu.CompilerParams(
            dimension_semantics=("arbitrary",)),
    )(xn, en, x_flat, embed)


_GCH = 128


def _sc_gather(table, idx):
    n = idx.shape[0]
    info = plsc.get_sparse_core_info()
    nw = info.num_cores * info.num_subcores
    per_w = n // nw
    mesh = plsc.VectorSubcoreMesh(core_axis_name="c", subcore_axis_name="s")

    @functools.partial(
        pl.kernel, mesh=mesh,
        out_type=jax.ShapeDtypeStruct((n, _D), table.dtype),
        scratch_types=[
            pltpu.VMEM((_GCH,), jnp.int32),
            pltpu.VMEM((_GCH, _D), jnp.float32),
            pltpu.SemaphoreType.DMA,
        ],
    )
    def gk(table_hbm, idx_hbm, out_hbm, idx_v, rows_v, sem):
        wid = lax.axis_index("s") * info.num_cores + lax.axis_index("c")
        base = wid * per_w

        @pl.loop(0, per_w // _GCH)
        def _(c):
            off = base + c * _GCH
            pltpu.sync_copy(idx_hbm.at[pl.ds(off, _GCH)], idx_v)
            pltpu.async_copy(table_hbm.at[idx_v], rows_v, sem).wait()
            pltpu.sync_copy(rows_v, out_hbm.at[pl.ds(off, _GCH)])

    return gk(table, idx)


def kernel(x, embed):
    shape = x.shape
    x_flat = x.reshape(-1, shape[-1])
    xn = jnp.sum(x_flat ** 2, axis=1, keepdims=True)
    embed_t = embed.T
    en = jnp.sum(embed_t ** 2, axis=0, keepdims=True)
    ind_flat = _nearest_code(x_flat, xn, en, embed).reshape(-1)
    quantized = _sc_gather(embed, ind_flat).reshape(shape)
    embed_ind = ind_flat.reshape(shape[:-1])
    return quantized, embed_ind
